# baseline (speedup 1.0000x reference)
import functools

import jax
import jax.numpy as jnp
from jax.experimental import pallas as pl
from jax.experimental.pallas import tpu as pltpu

B = 16384
K = 1024
D = 512
ALPHA = 1.0

BM = 4096
NB = B // BM
TM = 256
NT = BM // TM


def _loss_kernel(emb_ref, lab_ref, cent_ref, out_ref, cn_ref, b2_ref,
                 wb_ref, ws_ref, nn_ref):
    i = pl.program_id(0)

    @pl.when(i == 0)
    def _init():
        c = cent_ref[...]
        cs = jnp.sum(c * c, axis=1, keepdims=True)
        cnorm = c / jnp.maximum(jnp.sqrt(cs), 1e-12)
        cn_ref[...] = cnorm.T
        b2_ref[...] = jnp.sum(cnorm * cnorm, axis=1, keepdims=True).T
        nn_ref[...] = jnp.zeros((TM,), jnp.float32)

    cn = cn_ref[...]
    b2 = b2_ref[...]
    acc_n = nn_ref[...]
    for t in range(NT):
        e = emb_ref[t * TM:(t + 1) * TM, :]
        es = jnp.sum(e * e, axis=1, keepdims=True)
        en = e / jnp.maximum(jnp.sqrt(es), 1e-12)
        a2 = jnp.sum(en * en, axis=1, keepdims=True)
        ab = jax.lax.dot_general(
            en, cn, (((1,), (0,)), ((), ())),
            preferred_element_type=jnp.float32,
        )
        d2 = a2 + b2 - 2.0 * ab
        m2 = jnp.maximum(jnp.min(d2, axis=1), 1e-12)
        md = jnp.sqrt(m2)
        min_d2 = md * md

        lab = lab_ref[t * TM:(t + 1) * TM]
        bona = lab == 0
        wb_ref[pl.ds(i * BM + t * TM, TM)] = jnp.where(bona, min_d2, 0.0)
        ws_ref[pl.ds(i * BM + t * TM, TM)] = jnp.where(bona, 0.0, min_d2)
        acc_n = acc_n + lab.astype(jnp.float32)
    nn_ref[...] = acc_n

    @pl.when(i == NB - 1)
    def _finalize():
        n_spoof = jnp.sum(nn_ref[...])
        n_bona = float(B) - n_spoof
        bona_sum = jnp.sum(wb_ref[...])
        spoof_sum = jnp.sum(ws_ref[...])
        bona_loss = bona_sum / jnp.maximum(n_bona, 1.0)
        spoof_loss = -ALPHA * (spoof_sum / jnp.maximum(n_spoof, 1.0))
        total = (jnp.where(n_bona > 0.0, bona_loss, 0.0)
                 + jnp.where(n_spoof > 0.0, spoof_loss, 0.0))
        out_ref[0, 0] = total


@functools.partial(jax.jit, static_argnames=("interpret",))
def kernel(embeddings, labels, bonafide_centers, interpret=False):
    out = pl.pallas_call(
        _loss_kernel,
        grid=(NB,),
        in_specs=[
            pl.BlockSpec((BM, D), lambda i: (i, 0)),
            pl.BlockSpec((BM,), lambda i: (i,)),
            pl.BlockSpec((K, D), lambda i: (0, 0)),
        ],
        out_specs=pl.BlockSpec(memory_space=pltpu.SMEM),
        out_shape=jax.ShapeDtypeStruct((1, 1), jnp.float32),
        scratch_shapes=[pltpu.VMEM((D, K), jnp.float32),
                        pltpu.VMEM((1, K), jnp.float32),
                        pltpu.VMEM((B,), jnp.float32),
                        pltpu.VMEM((B,), jnp.float32),
                        pltpu.VMEM((TM,), jnp.float32)],
        interpret=interpret,
    )(embeddings, labels, bonafide_centers)
    return out[0, 0]
